# baseline (speedup 1.0000x reference)
import functools

import jax
import jax.numpy as jnp
from jax import lax
from jax.experimental import pallas as pl
from jax.experimental.pallas import tpu as pltpu
from jax.experimental.pallas import tpu_sc as plsc

_NC = 2
_NS = 16
_NW = _NC * _NS

_BATCH = 16384
_D = 64
_ENT_B = 4 * _BATCH
_REL_B = 2 * _BATCH
_EPW = _ENT_B // _NW
_RPW = _REL_B // _NW
_CH = 1024

_G = 8
_BB = _BATCH // _G


def _sc_gather(entities, relations, ent_idx, rel_idx):
    mesh = plsc.VectorSubcoreMesh(core_axis_name="c", subcore_axis_name="s")

    @functools.partial(
        pl.kernel,
        out_type=[
            jax.ShapeDtypeStruct((_ENT_B, _D), jnp.float32),
            jax.ShapeDtypeStruct((_REL_B, _D), jnp.float32),
        ],
        mesh=mesh,
        scratch_types=[
            pltpu.VMEM((_CH,), jnp.int32),
            pltpu.VMEM((_CH, _D), jnp.float32),
            pltpu.SemaphoreType.DMA,
        ],
        compiler_params=pltpu.CompilerParams(use_tc_tiling_on_sc=False),
    )
    def k(ent_hbm, rel_hbm, eidx_hbm, ridx_hbm, eout_hbm, rout_hbm,
          idx_v, rows_v, sem):
        wid = lax.axis_index("s") * _NC + lax.axis_index("c")

        ebase = wid * _EPW
        for i in range(_EPW // _CH):
            base = ebase + i * _CH
            pltpu.sync_copy(eidx_hbm.at[pl.ds(base, _CH)], idx_v)
            pltpu.async_copy(ent_hbm.at[idx_v], rows_v, sem).wait()
            pltpu.sync_copy(rows_v, eout_hbm.at[pl.ds(base, _CH)])

        rbase = wid * _RPW
        pltpu.sync_copy(ridx_hbm.at[pl.ds(rbase, _RPW)],
                        idx_v.at[pl.ds(0, _RPW)])
        pltpu.async_copy(rel_hbm.at[idx_v.at[pl.ds(0, _RPW)]],
                         rows_v.at[pl.ds(0, _RPW)], sem).wait()
        pltpu.sync_copy(rows_v.at[pl.ds(0, _RPW)],
                        rout_hbm.at[pl.ds(rbase, _RPW)])

    return k(entities, relations, ent_idx, rel_idx)


def _tc_loss_body(ph, pt, nh, nt, pr, nr, out_ref):
    i = pl.program_id(0)
    pdiff = ph[...] + pr[...] - pt[...]
    ndiff = nh[...] + nr[...] - nt[...]
    pd2 = jnp.sum(pdiff * pdiff, axis=1) + 1e-12
    nd2 = jnp.sum(ndiff * ndiff, axis=1) + 1e-12
    part = jnp.sum(jnp.maximum(jnp.sqrt(pd2) - jnp.sqrt(nd2) + 1.0, 0.0))

    @pl.when(i == 0)
    def _():
        out_ref[0, 0] = 0.0

    out_ref[0, 0] += part

    @pl.when(i == _G - 1)
    def _():
        out_ref[0, 0] = out_ref[0, 0] * (1.0 / _BATCH)


def _tc_loss(ent_rows, rel_rows, interpret=False):
    seg = _BATCH // _BB
    row_spec = lambda s: pl.BlockSpec((_BB, _D), lambda i, s=s: (s * seg + i, 0))
    out = pl.pallas_call(
        _tc_loss_body,
        grid=(_G,),
        in_specs=[row_spec(0), row_spec(1), row_spec(2), row_spec(3),
                  row_spec(0), row_spec(1)],
        out_specs=pl.BlockSpec((1, 1), lambda i: (0, 0),
                               memory_space=pltpu.SMEM),
        out_shape=jax.ShapeDtypeStruct((1, 1), jnp.float32),
        compiler_params=pltpu.CompilerParams(
            dimension_semantics=("arbitrary",)),
        interpret=interpret,
    )(ent_rows, ent_rows, ent_rows, ent_rows, rel_rows, rel_rows)
    return out[0, 0]


def kernel(positive_triples, negative_triples, entities, relations):
    pt32 = positive_triples.astype(jnp.int32)
    nt32 = negative_triples.astype(jnp.int32)
    ent_idx = jnp.concatenate([pt32[:, 0], pt32[:, 2], nt32[:, 0], nt32[:, 2]])
    rel_idx = jnp.concatenate([pt32[:, 1], nt32[:, 1]])
    ent_rows, rel_rows = _sc_gather(entities, relations, ent_idx, rel_idx)
    return _tc_loss(ent_rows, rel_rows)


# speedup vs baseline: 1.5862x; 1.5862x over previous
import functools

import jax
import jax.numpy as jnp
from jax import lax
from jax.experimental import pallas as pl
from jax.experimental.pallas import tpu as pltpu
from jax.experimental.pallas import tpu_sc as plsc

_NC = 2
_NS = 16
_NW = _NC * _NS

_BATCH = 16384
_D = 64
_ENT_B = 4 * _BATCH
_EPW = _ENT_B // _NW
_CH = 512
_SUB = 64

_G = 8
_BB = _BATCH // _G
_RK = 1024


def _sc_gather_ent(entities, ent_idx):
    mesh = plsc.VectorSubcoreMesh(core_axis_name="c", subcore_axis_name="s")

    @functools.partial(
        pl.kernel,
        out_type=jax.ShapeDtypeStruct((_ENT_B, _D), jnp.float32),
        mesh=mesh,
        scratch_types=[
            pltpu.VMEM((_CH,), jnp.int32),
            pltpu.VMEM((_CH, _D), jnp.float32),
            pltpu.SemaphoreType.DMA,
        ],
    )
    def k(ent_hbm, idx_hbm, out_hbm, idx_v, out_v, sem):
        wid = lax.axis_index("s") * _NC + lax.axis_index("c")
        wbase = wid * _EPW

        for ch in range(_EPW // _CH):
            base = wbase + ch * _CH
            pltpu.sync_copy(idx_hbm.at[pl.ds(base, _CH)], idx_v)

            @pl.loop(0, _CH // _SUB)
            def _(sc):
                sb = sc * _SUB
                copies = []
                for g in range(_SUB // 16):
                    vec = idx_v[pl.ds(sb + g * 16, 16)]
                    for j in range(16):
                        copies.append(pltpu.async_copy(
                            ent_hbm.at[vec[j]], out_v.at[sb + g * 16 + j],
                            sem))
                for c in copies:
                    c.wait()

            pltpu.sync_copy(out_v, out_hbm.at[pl.ds(base, _CH)])

    return k(entities, ent_idx)


def _tc_loss_body(ph, pt, nh, nt, pri, nri, tab, out_ref):
    i = pl.program_id(0)
    iota = jax.lax.broadcasted_iota(jnp.int32, (_BB, _RK), 1)
    oh_p = jnp.where(iota == pri[...], 1.0, 0.0).astype(jnp.float32)
    oh_n = jnp.where(iota == nri[...], 1.0, 0.0).astype(jnp.float32)
    pr = jnp.dot(oh_p, tab[...], preferred_element_type=jnp.float32)
    nr = jnp.dot(oh_n, tab[...], preferred_element_type=jnp.float32)

    pdiff = ph[...] + pr - pt[...]
    ndiff = nh[...] + nr - nt[...]
    pd2 = jnp.sum(pdiff * pdiff, axis=1) + 1e-12
    nd2 = jnp.sum(ndiff * ndiff, axis=1) + 1e-12
    part = jnp.sum(jnp.maximum(jnp.sqrt(pd2) - jnp.sqrt(nd2) + 1.0, 0.0))

    @pl.when(i == 0)
    def _():
        out_ref[0, 0] = 0.0

    out_ref[0, 0] += part

    @pl.when(i == _G - 1)
    def _():
        out_ref[0, 0] = out_ref[0, 0] * (1.0 / _BATCH)


def _tc_loss(ent_rows, pr_idx, nr_idx, rel_pad, interpret=False):
    seg = _BATCH // _BB
    row_spec = lambda s: pl.BlockSpec((_BB, _D), lambda i, s=s: (s * seg + i, 0))
    idx_spec = pl.BlockSpec((_BB, 1), lambda i: (i, 0))
    tab_spec = pl.BlockSpec((_RK, _D), lambda i: (0, 0))
    out = pl.pallas_call(
        _tc_loss_body,
        grid=(_G,),
        in_specs=[row_spec(0), row_spec(1), row_spec(2), row_spec(3),
                  idx_spec, idx_spec, tab_spec],
        out_specs=pl.BlockSpec((1, 1), lambda i: (0, 0),
                               memory_space=pltpu.SMEM),
        out_shape=jax.ShapeDtypeStruct((1, 1), jnp.float32),
        compiler_params=pltpu.CompilerParams(
            dimension_semantics=("arbitrary",)),
        interpret=interpret,
    )(ent_rows, ent_rows, ent_rows, ent_rows, pr_idx, nr_idx, rel_pad)
    return out[0, 0]


def kernel(positive_triples, negative_triples, entities, relations):
    pt32 = positive_triples.astype(jnp.int32)
    nt32 = negative_triples.astype(jnp.int32)
    ent_idx = jnp.concatenate([pt32[:, 0], pt32[:, 2], nt32[:, 0], nt32[:, 2]])
    ent_rows = _sc_gather_ent(entities, ent_idx)

    pr_idx = pt32[:, 1:2]
    nr_idx = nt32[:, 1:2]
    rel_pad = jnp.pad(relations, ((0, _RK - relations.shape[0]), (0, 0)))
    return _tc_loss(ent_rows, pr_idx, nr_idx, rel_pad)
